# Initial kernel scaffold; baseline (speedup 1.0000x reference)
import jax
import jax.numpy as jnp
from jax.experimental import pallas as pl


def kernel(x, emb0, emb1, emb2, W1, W2):
    raise NotImplementedError("write your pallas kernel here")


# speedup vs baseline: 5.6295x; 5.6295x over previous
import functools

import jax
import jax.numpy as jnp
from jax import lax
from jax.experimental import pallas as pl
from jax.experimental.pallas import tpu as pltpu
from jax.experimental.pallas import tpu_sc as plsc

VOCAB = 1_000_000
D_MODEL = 128
C0 = 20_000
C1 = 40_000
N_TOK = 4096 * 50
NC, NS, L = 2, 16, 16
NW = NC * NS
TPW = N_TOK // NW
G = 128
NGRP = TPW // G
NCGRP = TPW // G + 1

_SC_MESH = plsc.VectorSubcoreMesh(
    core_axis_name="c", subcore_axis_name="s", num_cores=NC, num_subcores=NS
)


@functools.partial(
    pl.kernel,
    out_type=jax.ShapeDtypeStruct((N_TOK, 8), jnp.float32),
    mesh=_SC_MESH,
    scratch_types=[
        pltpu.VMEM((TPW,), jnp.int32),
        pltpu.VMEM((NGRP, G), jnp.int32),
        pltpu.VMEM((TPW, 8), jnp.float32),
        pltpu.SemaphoreType.DMA,
    ],
    compiler_params=pltpu.CompilerParams(use_tc_tiling_on_sc=False, needs_layout_passes=False),
)
def _sc_gather_e2(x_hbm, emb2_hbm, e2buf_hbm, xv, idxv, rows, sem):
    wid = lax.axis_index("s") * NC + lax.axis_index("c")
    base = wid * TPW
    pltpu.sync_copy(x_hbm.at[pl.ds(base, TPW)], xv)

    @pl.loop(0, TPW // L)
    def _(g):
        xs = xv[pl.ds(g * L, L)]
        idxv[g // (G // L), pl.ds((g % (G // L)) * L, L)] = jnp.clip(
            xs - C1, 0, VOCAB - C1 - 1
        )

    @pl.loop(0, NGRP)
    def _(j):
        pltpu.async_copy(emb2_hbm.at[idxv.at[j]], rows.at[pl.ds(j * G, G)], sem)

    @pl.loop(0, NGRP)
    def _(j):
        pltpu.make_async_copy(
            emb2_hbm.at[idxv.at[j]], rows.at[pl.ds(j * G, G)], sem
        ).wait()

    pltpu.sync_copy(rows, e2buf_hbm.at[pl.ds(base, TPW)])


_TROWS = 400


def _build_t_body(emb0_ref, emb1_ref, w1_ref, t_ref):
    g = pl.program_id(0)

    @pl.when(g < 50)
    def _():
        t_ref[...] = emb0_ref[...]

    @pl.when(g >= 50)
    def _():
        t_ref[...] = jnp.dot(
            emb1_ref[...],
            w1_ref[...],
            preferred_element_type=jnp.float32,
            precision=lax.Precision.HIGHEST,
        )


def _build_t(emb0, emb1, W1):
    return pl.pallas_call(
        _build_t_body,
        grid=(100,),
        in_specs=[
            pl.BlockSpec((_TROWS, 128), lambda g: (jnp.minimum(g, 49), 0)),
            pl.BlockSpec((_TROWS, 32), lambda g: (jnp.maximum(g - 50, 0), 0)),
            pl.BlockSpec((32, 128), lambda g: (0, 0)),
        ],
        out_specs=pl.BlockSpec((_TROWS, 128), lambda g: (g, 0)),
        out_shape=jax.ShapeDtypeStruct((C1, 128), jnp.float32),
    )(emb0, emb1, W1)


_MROWS = 2048


def _mm_body(e2_ref, w2_ref, o_ref):
    o_ref[...] = jnp.dot(
        e2_ref[...],
        w2_ref[...],
        preferred_element_type=jnp.float32,
        precision=lax.Precision.HIGHEST,
    )


def _mm_e2(e2buf, W2):
    return pl.pallas_call(
        _mm_body,
        grid=(N_TOK // _MROWS,),
        in_specs=[
            pl.BlockSpec((_MROWS, 8), lambda g: (g, 0)),
            pl.BlockSpec((8, 128), lambda g: (0, 0)),
        ],
        out_specs=pl.BlockSpec((_MROWS, 128), lambda g: (g, 0)),
        out_shape=jax.ShapeDtypeStruct((N_TOK, 128), jnp.float32),
    )(e2buf, W2)


@functools.partial(
    pl.kernel,
    out_type=(),
    mesh=_SC_MESH,
    scratch_types=[
        pltpu.VMEM((TPW,), jnp.int32),
        pltpu.VMEM((NCGRP, G), jnp.int32),
        pltpu.VMEM((NCGRP, G), jnp.int32),
        pltpu.VMEM((G, 128), jnp.float32),
        pltpu.SemaphoreType.DMA,
        pltpu.SemaphoreType.DMA,
    ],
    compiler_params=pltpu.CompilerParams(use_tc_tiling_on_sc=False, needs_layout_passes=False),
)
def _sc_scatter_t(x_hbm, t_hbm, out_hbm, xv, srcc, dstc, rows, gsem, ssem):
    wid = lax.axis_index("s") * NC + lax.axis_index("c")
    base = wid * TPW
    pltpu.sync_copy(x_hbm.at[pl.ds(base, TPW)], xv)

    def _compact(g, cnt):
        xs = xv[pl.ds(g * L, L)]
        m = xs < C1
        mi = jnp.where(m, 1, 0).astype(jnp.int32)
        pos = cnt + plsc.cumsum(mi) - 1
        src = jnp.minimum(xs, C1 - 1)
        dst = base + g * L + lax.iota(jnp.int32, L)
        plsc.store_scatter(srcc, [pos // G, pos % G], src, mask=m)
        plsc.store_scatter(dstc, [pos // G, pos % G], dst, mask=m)
        return cnt + jnp.sum(mi)

    n = lax.fori_loop(0, TPW // L, _compact, jnp.int32(0))

    lastp = jnp.maximum(n - 1, 0)
    lrow = jnp.full((L,), lastp // G, jnp.int32)
    lcol = jnp.full((L,), lastp % G, jnp.int32)
    lastsrc = plsc.load_gather(srcc, [lrow, lcol])
    lastdst = plsc.load_gather(dstc, [lrow, lcol])

    @pl.loop(0, G // L)
    def _(k):
        p = n + k * L + lax.iota(jnp.int32, L)
        plsc.store_scatter(srcc, [p // G, p % G], lastsrc)
        plsc.store_scatter(dstc, [p // G, p % G], lastdst)

    nsg = (n + G - 1) // G

    def _dma(j, carry):
        pltpu.async_copy(t_hbm.at[srcc.at[j]], rows, gsem)
        pltpu.make_async_copy(t_hbm.at[srcc.at[j]], rows, gsem).wait()
        pltpu.async_copy(rows, out_hbm.at[dstc.at[j]], ssem)
        pltpu.make_async_copy(rows, out_hbm.at[dstc.at[j]], ssem).wait()
        return carry

    lax.fori_loop(0, nsg, _dma, jnp.int32(0))


def kernel(x, emb0, emb1, emb2, W1, W2):
    xf = x.reshape(-1)
    e2buf = _sc_gather_e2(xf, emb2)
    t = _build_t(emb0, emb1, W1)
    out0 = _mm_e2(e2buf, W2)
    out_ref = jax.new_ref(out0)
    _sc_scatter_t(xf, t, out_ref)
    return out_ref[...].reshape(x.shape[0], x.shape[1], D_MODEL)


# speedup vs baseline: 7.2272x; 1.2838x over previous
import functools

import jax
import jax.numpy as jnp
from jax import lax
from jax.experimental import pallas as pl
from jax.experimental.pallas import tpu as pltpu
from jax.experimental.pallas import tpu_sc as plsc

VOCAB = 1_000_000
D_MODEL = 128
C0 = 20_000
C1 = 40_000
N_TOK = 4096 * 50
NC, NS, L = 2, 16, 16
NW = NC * NS
TPW = N_TOK // NW
G = 128
NGRP = TPW // G
NCGRP = TPW // G + 1

_SC_MESH = plsc.VectorSubcoreMesh(
    core_axis_name="c", subcore_axis_name="s", num_cores=NC, num_subcores=NS
)


@functools.partial(
    pl.kernel,
    out_type=jax.ShapeDtypeStruct((N_TOK, 8), jnp.float32),
    mesh=_SC_MESH,
    scratch_types=[
        pltpu.VMEM((TPW,), jnp.int32),
        pltpu.VMEM((NGRP, G), jnp.int32),
        pltpu.VMEM((TPW, 8), jnp.float32),
        pltpu.SemaphoreType.DMA,
    ],
    compiler_params=pltpu.CompilerParams(use_tc_tiling_on_sc=False, needs_layout_passes=False),
)
def _sc_gather_e2(x_hbm, emb2_hbm, e2buf_hbm, xv, idxv, rows, sem):
    wid = lax.axis_index("s") * NC + lax.axis_index("c")
    base = wid * TPW
    pltpu.sync_copy(x_hbm.at[pl.ds(base, TPW)], xv)

    @pl.loop(0, TPW // L)
    def _(g):
        xs = xv[pl.ds(g * L, L)]
        idxv[g // (G // L), pl.ds((g % (G // L)) * L, L)] = jnp.clip(
            xs - C1, 0, VOCAB - C1 - 1
        )

    @pl.loop(0, NGRP)
    def _(j):
        pltpu.async_copy(emb2_hbm.at[idxv.at[j]], rows.at[pl.ds(j * G, G)], sem)

    @pl.loop(0, NGRP)
    def _(j):
        pltpu.make_async_copy(
            emb2_hbm.at[idxv.at[j]], rows.at[pl.ds(j * G, G)], sem
        ).wait()

    pltpu.sync_copy(rows, e2buf_hbm.at[pl.ds(base, TPW)])


_TROWS = 400


def _build_t_body(emb0_ref, emb1_ref, w1_ref, t_ref):
    g = pl.program_id(0)

    @pl.when(g < 50)
    def _():
        t_ref[...] = emb0_ref[...]

    @pl.when(g >= 50)
    def _():
        t_ref[...] = jnp.dot(
            emb1_ref[...],
            w1_ref[...],
            preferred_element_type=jnp.float32,
            precision=lax.Precision.HIGHEST,
        )


def _build_t(emb0, emb1, W1):
    return pl.pallas_call(
        _build_t_body,
        grid=(100,),
        in_specs=[
            pl.BlockSpec((_TROWS, 128), lambda g: (jnp.minimum(g, 49), 0)),
            pl.BlockSpec((_TROWS, 32), lambda g: (jnp.maximum(g - 50, 0), 0)),
            pl.BlockSpec((32, 128), lambda g: (0, 0)),
        ],
        out_specs=pl.BlockSpec((_TROWS, 128), lambda g: (g, 0)),
        out_shape=jax.ShapeDtypeStruct((C1, 128), jnp.float32),
    )(emb0, emb1, W1)


_MROWS = 2048


def _mm_body(e2_ref, w2_ref, o_ref):
    o_ref[...] = jnp.dot(
        e2_ref[...],
        w2_ref[...],
        preferred_element_type=jnp.float32,
        precision=lax.Precision.HIGHEST,
    )


def _mm_e2(e2buf, W2):
    return pl.pallas_call(
        _mm_body,
        grid=(N_TOK // _MROWS,),
        in_specs=[
            pl.BlockSpec((_MROWS, 8), lambda g: (g, 0)),
            pl.BlockSpec((8, 128), lambda g: (0, 0)),
        ],
        out_specs=pl.BlockSpec((_MROWS, 128), lambda g: (g, 0)),
        out_shape=jax.ShapeDtypeStruct((N_TOK, 128), jnp.float32),
    )(e2buf, W2)


@functools.partial(
    pl.kernel,
    out_type=(),
    mesh=_SC_MESH,
    scratch_types=[
        pltpu.VMEM((TPW,), jnp.int32),
        pltpu.VMEM((NCGRP, G), jnp.int32),
        pltpu.VMEM((NCGRP, G), jnp.int32),
        pltpu.VMEM((G, 128), jnp.float32),
        pltpu.SemaphoreType.DMA,
        pltpu.SemaphoreType.DMA,
    ],
    compiler_params=pltpu.CompilerParams(use_tc_tiling_on_sc=False, needs_layout_passes=False),
)
def _sc_scatter_t(x_hbm, t_hbm, out_hbm, xv, srcc, dstc, rows, gsem, ssem):
    wid = lax.axis_index("s") * NC + lax.axis_index("c")
    base = wid * TPW
    pltpu.sync_copy(x_hbm.at[pl.ds(base, TPW)], xv)

    def _compact(g, cnt):
        xs = xv[pl.ds(g * L, L)]
        m = xs < C1
        mi = jnp.where(m, 1, 0).astype(jnp.int32)
        pos = cnt + plsc.cumsum(mi) - 1
        src = jnp.minimum(xs, C1 - 1)
        dst = base + g * L + lax.iota(jnp.int32, L)
        plsc.store_scatter(srcc, [pos // G, pos % G], src, mask=m)
        plsc.store_scatter(dstc, [pos // G, pos % G], dst, mask=m)
        return cnt + jnp.sum(mi)

    n = lax.fori_loop(0, TPW // L, _compact, jnp.int32(0))

    lastp = jnp.maximum(n - 1, 0)
    lrow = jnp.full((L,), lastp // G, jnp.int32)
    lcol = jnp.full((L,), lastp % G, jnp.int32)
    lastsrc = plsc.load_gather(srcc, [lrow, lcol])
    lastdst = plsc.load_gather(dstc, [lrow, lcol])

    @pl.loop(0, G // L)
    def _(k):
        p = n + k * L + lax.iota(jnp.int32, L)
        plsc.store_scatter(srcc, [p // G, p % G], lastsrc)
        plsc.store_scatter(dstc, [p // G, p % G], lastdst)

    nsg = (n + G - 1) // G

    def _dma(j, carry):
        pltpu.async_copy(t_hbm.at[srcc.at[j]], rows, gsem)
        pltpu.make_async_copy(t_hbm.at[srcc.at[j]], rows, gsem).wait()
        pltpu.async_copy(rows, out_hbm.at[dstc.at[j]], ssem)
        pltpu.make_async_copy(rows, out_hbm.at[dstc.at[j]], ssem).wait()
        return carry

    lax.fori_loop(0, nsg, _dma, jnp.int32(0))


def kernel(x, emb0, emb1, emb2, W1, W2):
    b, s = x.shape
    xp = x.T.reshape(-1)
    e2buf = _sc_gather_e2(xp, emb2)
    t = _build_t(emb0, emb1, W1)
    out0 = _mm_e2(e2buf, W2)
    out_ref = jax.new_ref(out0)
    _sc_scatter_t(xp, t, out_ref)
    return out_ref[...].reshape(s, b, D_MODEL).transpose(1, 0, 2)


# speedup vs baseline: 11.2038x; 1.5502x over previous
import functools

import jax
import jax.numpy as jnp
from jax import lax
from jax.experimental import pallas as pl
from jax.experimental.pallas import tpu as pltpu
from jax.experimental.pallas import tpu_sc as plsc

VOCAB = 1_000_000
D_MODEL = 128
C0 = 20_000
C1 = 40_000
N_TOK = 4096 * 50
NC, NS, L = 2, 16, 16
NW = NC * NS
TPW = N_TOK // NW
G = 128
NGRP = TPW // G
NCGRP = TPW // G + 1

_SC_MESH = plsc.VectorSubcoreMesh(
    core_axis_name="c", subcore_axis_name="s", num_cores=NC, num_subcores=NS
)


@functools.partial(
    pl.kernel,
    out_type=jax.ShapeDtypeStruct((N_TOK, 8), jnp.float32),
    mesh=_SC_MESH,
    scratch_types=[
        pltpu.VMEM((TPW,), jnp.int32),
        pltpu.VMEM((NGRP, G), jnp.int32),
        pltpu.VMEM((TPW, 8), jnp.float32),
        pltpu.SemaphoreType.DMA,
    ],
    compiler_params=pltpu.CompilerParams(use_tc_tiling_on_sc=False, needs_layout_passes=False),
)
def _sc_gather_e2(x_hbm, emb2_hbm, e2buf_hbm, xv, idxv, rows, sem):
    wid = lax.axis_index("s") * NC + lax.axis_index("c")
    base = wid * TPW
    pltpu.sync_copy(x_hbm.at[pl.ds(base, TPW)], xv)

    @pl.loop(0, TPW // L)
    def _(g):
        xs = xv[pl.ds(g * L, L)]
        idxv[g // (G // L), pl.ds((g % (G // L)) * L, L)] = jnp.clip(
            xs - C1, 0, VOCAB - C1 - 1
        )

    @pl.loop(0, NGRP)
    def _(j):
        pltpu.async_copy(emb2_hbm.at[idxv.at[j]], rows.at[pl.ds(j * G, G)], sem)

    @pl.loop(0, NGRP)
    def _(j):
        pltpu.make_async_copy(
            emb2_hbm.at[idxv.at[j]], rows.at[pl.ds(j * G, G)], sem
        ).wait()

    pltpu.sync_copy(rows, e2buf_hbm.at[pl.ds(base, TPW)])


_PT = 25
_NSLAB = 7500 // _PT


@functools.partial(
    pl.kernel,
    out_type=jax.ShapeDtypeStruct((VOCAB - C1, 8), jnp.float32),
    mesh=_SC_MESH,
    scratch_types=[
        pltpu.VMEM((8 * _PT, 128), jnp.float32),
        pltpu.VMEM((128 * _PT, 8), jnp.float32),
    ],
    compiler_params=pltpu.CompilerParams(use_tc_tiling_on_sc=False, needs_layout_passes=False),
)
def _sc_pack_e2(v_hbm, o_hbm, vin, vout):
    wid = lax.axis_index("s") * NC + lax.axis_index("c")
    nmy = (_NSLAB - 1 - wid) // NW + 1
    lanes = lax.iota(jnp.int32, L)
    comp = lanes % 8
    tok2 = lanes // 8

    def _slab(i, carry):
        s = wid + i * NW
        pltpu.sync_copy(v_hbm.at[pl.ds(s * 8 * _PT, 8 * _PT)], vin)

        @pl.loop(0, _PT)
        def _(t):
            @pl.loop(0, 64)
            def _(u):
                c0 = (u // 8) * 16 + (u % 8) * 2
                val = plsc.load_gather(vin, [t * 8 + comp, tok2 + c0])
                plsc.store_scatter(vout, [t * 128 + c0 + tok2, comp], val)

        pltpu.sync_copy(vout, o_hbm.at[pl.ds(s * 128 * _PT, 128 * _PT)])
        return carry

    lax.fori_loop(0, nmy, _slab, jnp.int32(0))


_TROWS = 400


def _build_t_body(emb0_ref, emb1_ref, w1_ref, t_ref):
    g = pl.program_id(0)

    @pl.when(g < 50)
    def _():
        t_ref[...] = emb0_ref[...]

    @pl.when(g >= 50)
    def _():
        t_ref[...] = jnp.dot(
            emb1_ref[...],
            w1_ref[...],
            preferred_element_type=jnp.float32,
            precision=lax.Precision.HIGHEST,
        )


def _build_t(emb0, emb1, W1):
    return pl.pallas_call(
        _build_t_body,
        grid=(100,),
        in_specs=[
            pl.BlockSpec((_TROWS, 128), lambda g: (jnp.minimum(g, 49), 0)),
            pl.BlockSpec((_TROWS, 32), lambda g: (jnp.maximum(g - 50, 0), 0)),
            pl.BlockSpec((32, 128), lambda g: (0, 0)),
        ],
        out_specs=pl.BlockSpec((_TROWS, 128), lambda g: (g, 0)),
        out_shape=jax.ShapeDtypeStruct((C1, 128), jnp.float32),
    )(emb0, emb1, W1)


_MROWS = 512


def _mm_body(p_ref, b_ref, o_ref):
    o_ref[...] = jnp.dot(
        p_ref[...],
        b_ref[...],
        preferred_element_type=jnp.float32,
        precision=lax.Precision.HIGHEST,
    )


def _mm_e2(e2packed, Bmat):
    return pl.pallas_call(
        _mm_body,
        grid=(N_TOK // 16 // _MROWS,),
        in_specs=[
            pl.BlockSpec((_MROWS, 128), lambda g: (g, 0)),
            pl.BlockSpec((128, 2048), lambda g: (0, 0)),
        ],
        out_specs=pl.BlockSpec((_MROWS, 2048), lambda g: (g, 0)),
        out_shape=jax.ShapeDtypeStruct((N_TOK // 16, 2048), jnp.float32),
    )(e2packed, Bmat)


@functools.partial(
    pl.kernel,
    out_type=(),
    mesh=_SC_MESH,
    scratch_types=[
        pltpu.VMEM((TPW,), jnp.int32),
        pltpu.VMEM((NCGRP, G), jnp.int32),
        pltpu.VMEM((NCGRP, G), jnp.int32),
        pltpu.VMEM((G, 128), jnp.float32),
        pltpu.SemaphoreType.DMA,
        pltpu.SemaphoreType.DMA,
    ],
    compiler_params=pltpu.CompilerParams(use_tc_tiling_on_sc=False, needs_layout_passes=False),
)
def _sc_scatter_t(x_hbm, t_hbm, out_hbm, xv, srcc, dstc, rows, gsem, ssem):
    wid = lax.axis_index("s") * NC + lax.axis_index("c")
    base = wid * TPW
    pltpu.sync_copy(x_hbm.at[pl.ds(base, TPW)], xv)

    def _compact(g, cnt):
        xs = xv[pl.ds(g * L, L)]
        m = xs < C1
        mi = jnp.where(m, 1, 0).astype(jnp.int32)
        pos = cnt + plsc.cumsum(mi) - 1
        src = jnp.minimum(xs, C1 - 1)
        dst = base + g * L + lax.iota(jnp.int32, L)
        plsc.store_scatter(srcc, [pos // G, pos % G], src, mask=m)
        plsc.store_scatter(dstc, [pos // G, pos % G], dst, mask=m)
        return cnt + jnp.sum(mi)

    n = lax.fori_loop(0, TPW // L, _compact, jnp.int32(0))

    lastp = jnp.maximum(n - 1, 0)
    lrow = jnp.full((L,), lastp // G, jnp.int32)
    lcol = jnp.full((L,), lastp % G, jnp.int32)
    lastsrc = plsc.load_gather(srcc, [lrow, lcol])
    lastdst = plsc.load_gather(dstc, [lrow, lcol])

    @pl.loop(0, G // L)
    def _(k):
        p = n + k * L + lax.iota(jnp.int32, L)
        plsc.store_scatter(srcc, [p // G, p % G], lastsrc)
        plsc.store_scatter(dstc, [p // G, p % G], lastdst)

    nsg = (n + G - 1) // G

    def _dma(j, carry):
        pltpu.async_copy(t_hbm.at[srcc.at[j]], rows, gsem)
        pltpu.make_async_copy(t_hbm.at[srcc.at[j]], rows, gsem).wait()
        pltpu.async_copy(rows, out_hbm.at[dstc.at[j]], ssem)
        pltpu.make_async_copy(rows, out_hbm.at[dstc.at[j]], ssem).wait()
        return carry

    lax.fori_loop(0, nsg, _dma, jnp.int32(0))


def kernel(x, emb0, emb1, emb2, W1, W2):
    b, s = x.shape
    xp = x.T.reshape(-1)
    v = emb2.reshape(7500, 128, 8).transpose(0, 2, 1).reshape(60000, 128)
    e2lin = _sc_pack_e2(v)
    e2buf = _sc_gather_e2(xp, e2lin)
    t = _build_t(emb0, emb1, W1)
    Bmat = (
        jnp.eye(16, dtype=jnp.float32)[:, None, :, None] * W2[None, :, None, :]
    ).reshape(128, 2048)
    out0 = _mm_e2(e2buf.reshape(N_TOK // 16, 128), Bmat).reshape(N_TOK, D_MODEL)
    out_ref = jax.new_ref(out0)
    _sc_scatter_t(xp, t, out_ref)
    return out_ref[...].reshape(s, b, D_MODEL).transpose(1, 0, 2)


# speedup vs baseline: 13.7336x; 1.2258x over previous
import functools

import jax
import jax.numpy as jnp
from jax import lax
from jax.experimental import pallas as pl
from jax.experimental.pallas import tpu as pltpu
from jax.experimental.pallas import tpu_sc as plsc

VOCAB = 1_000_000
D_MODEL = 128
C0 = 20_000
C1 = 40_000
N_TOK = 4096 * 50
NC, NS, L = 2, 16, 16
NW = NC * NS
TPW = N_TOK // NW
G = 128
NGRP = TPW // G
NCGRP = TPW // G + 1

_SC_MESH = plsc.VectorSubcoreMesh(
    core_axis_name="c", subcore_axis_name="s", num_cores=NC, num_subcores=NS
)


@functools.partial(
    pl.kernel,
    out_type=jax.ShapeDtypeStruct((N_TOK, 8), jnp.float32),
    mesh=_SC_MESH,
    scratch_types=[
        pltpu.VMEM((TPW,), jnp.int32),
        pltpu.VMEM((NGRP, G), jnp.int32),
        pltpu.VMEM((TPW, 8), jnp.float32),
        pltpu.SemaphoreType.DMA,
    ],
    compiler_params=pltpu.CompilerParams(use_tc_tiling_on_sc=False, needs_layout_passes=False),
)
def _sc_gather_e2(x_hbm, emb2_hbm, e2buf_hbm, xv, idxv, rows, sem):
    wid = lax.axis_index("s") * NC + lax.axis_index("c")
    base = wid * TPW
    pltpu.sync_copy(x_hbm.at[pl.ds(base, TPW)], xv)

    lanes = lax.iota(jnp.int32, L)
    qbase = 16 * (lanes % 8) + lanes // 8

    @pl.loop(0, TPW // L)
    def _(g):
        xs = xv[pl.ds(g * L, L)]
        grp = jnp.full((L,), g // (G // L), jnp.int32)
        q = qbase + 2 * (g % (G // L))
        plsc.store_scatter(idxv, [grp, q], jnp.clip(xs - C1, 0, VOCAB - C1 - 1))

    @pl.loop(0, NGRP)
    def _(j):
        pltpu.async_copy(emb2_hbm.at[idxv.at[j]], rows.at[pl.ds(j * G, G)], sem)

    @pl.loop(0, NGRP)
    def _(j):
        pltpu.make_async_copy(
            emb2_hbm.at[idxv.at[j]], rows.at[pl.ds(j * G, G)], sem
        ).wait()

    pltpu.sync_copy(rows, e2buf_hbm.at[pl.ds(base, TPW)])


_PT = 25
_NSLAB = 7500 // _PT


@functools.partial(
    pl.kernel,
    out_type=jax.ShapeDtypeStruct((VOCAB - C1, 8), jnp.float32),
    mesh=_SC_MESH,
    scratch_types=[
        pltpu.VMEM((8 * _PT, 128), jnp.float32),
        pltpu.VMEM((128 * _PT, 8), jnp.float32),
    ],
    compiler_params=pltpu.CompilerParams(use_tc_tiling_on_sc=False, needs_layout_passes=False),
)
def _sc_pack_e2(v_hbm, o_hbm, vin, vout):
    wid = lax.axis_index("s") * NC + lax.axis_index("c")
    nmy = (_NSLAB - 1 - wid) // NW + 1
    lanes = lax.iota(jnp.int32, L)
    comp = lanes % 8
    tok2 = lanes // 8

    def _slab(i, carry):
        s = wid + i * NW
        pltpu.sync_copy(v_hbm.at[pl.ds(s * 8 * _PT, 8 * _PT)], vin)

        @pl.loop(0, _PT)
        def _(t):
            @pl.loop(0, 64)
            def _(u):
                c0 = (u // 8) * 16 + (u % 8) * 2
                val = plsc.load_gather(vin, [t * 8 + comp, tok2 + c0])
                plsc.store_scatter(vout, [t * 128 + c0 + tok2, comp], val)

        pltpu.sync_copy(vout, o_hbm.at[pl.ds(s * 128 * _PT, 128 * _PT)])
        return carry

    lax.fori_loop(0, nmy, _slab, jnp.int32(0))


_TROWS = 400


def _build_t_body(emb0_ref, emb1_ref, w1_ref, t_ref):
    g = pl.program_id(0)

    @pl.when(g < 50)
    def _():
        t_ref[...] = emb0_ref[...]

    @pl.when(g >= 50)
    def _():
        t_ref[...] = jnp.dot(
            emb1_ref[...],
            w1_ref[...],
            preferred_element_type=jnp.float32,
            precision=lax.Precision.HIGHEST,
        )


def _build_t(emb0, emb1, W1):
    return pl.pallas_call(
        _build_t_body,
        grid=(100,),
        in_specs=[
            pl.BlockSpec((_TROWS, 128), lambda g: (jnp.minimum(g, 49), 0)),
            pl.BlockSpec((_TROWS, 32), lambda g: (jnp.maximum(g - 50, 0), 0)),
            pl.BlockSpec((32, 128), lambda g: (0, 0)),
        ],
        out_specs=pl.BlockSpec((_TROWS, 128), lambda g: (g, 0)),
        out_shape=jax.ShapeDtypeStruct((C1, 128), jnp.float32),
    )(emb0, emb1, W1)


_MROWS = 512


def _mm_body(p_ref, b_ref, o_ref):
    o_ref[...] = jnp.dot(
        p_ref[...],
        b_ref[...],
        preferred_element_type=jnp.float32,
        precision=lax.Precision.HIGHEST,
    )


def _mm_e2(e2packed, Bmat):
    return pl.pallas_call(
        _mm_body,
        grid=(N_TOK // 16 // _MROWS,),
        in_specs=[
            pl.BlockSpec((_MROWS, 128), lambda g: (g, 0)),
            pl.BlockSpec((128, 2048), lambda g: (0, 0)),
        ],
        out_specs=pl.BlockSpec((_MROWS, 2048), lambda g: (g, 0)),
        out_shape=jax.ShapeDtypeStruct((N_TOK // 16, 2048), jnp.float32),
    )(e2packed, Bmat)


@functools.partial(
    pl.kernel,
    out_type=(),
    mesh=_SC_MESH,
    scratch_types=[
        pltpu.VMEM((TPW,), jnp.int32),
        pltpu.VMEM((NCGRP, G), jnp.int32),
        pltpu.VMEM((NCGRP, G), jnp.int32),
        pltpu.VMEM((G, 128), jnp.float32),
        pltpu.SemaphoreType.DMA,
        pltpu.SemaphoreType.DMA,
    ],
    compiler_params=pltpu.CompilerParams(use_tc_tiling_on_sc=False, needs_layout_passes=False),
)
def _sc_scatter_t(x_hbm, t_hbm, out_hbm, xv, srcc, dstc, rows, gsem, ssem):
    wid = lax.axis_index("s") * NC + lax.axis_index("c")
    base = wid * TPW
    pltpu.sync_copy(x_hbm.at[pl.ds(base, TPW)], xv)

    def _compact(g, cnt):
        xs = xv[pl.ds(g * L, L)]
        m = xs < C1
        mi = jnp.where(m, 1, 0).astype(jnp.int32)
        pos = cnt + plsc.cumsum(mi) - 1
        src = jnp.minimum(xs, C1 - 1)
        dst = base + g * L + lax.iota(jnp.int32, L)
        plsc.store_scatter(srcc, [pos // G, pos % G], src, mask=m)
        plsc.store_scatter(dstc, [pos // G, pos % G], dst, mask=m)
        return cnt + jnp.sum(mi)

    n = lax.fori_loop(0, TPW // L, _compact, jnp.int32(0))

    lastp = jnp.maximum(n - 1, 0)
    lrow = jnp.full((L,), lastp // G, jnp.int32)
    lcol = jnp.full((L,), lastp % G, jnp.int32)
    lastsrc = plsc.load_gather(srcc, [lrow, lcol])
    lastdst = plsc.load_gather(dstc, [lrow, lcol])

    @pl.loop(0, G // L)
    def _(k):
        p = n + k * L + lax.iota(jnp.int32, L)
        plsc.store_scatter(srcc, [p // G, p % G], lastsrc)
        plsc.store_scatter(dstc, [p // G, p % G], lastdst)

    nsg = (n + G - 1) // G

    def _dma(j, carry):
        pltpu.async_copy(t_hbm.at[srcc.at[j]], rows, gsem)
        pltpu.make_async_copy(t_hbm.at[srcc.at[j]], rows, gsem).wait()
        pltpu.async_copy(rows, out_hbm.at[dstc.at[j]], ssem)
        pltpu.make_async_copy(rows, out_hbm.at[dstc.at[j]], ssem).wait()
        return carry

    lax.fori_loop(0, nsg, _dma, jnp.int32(0))


def kernel(x, emb0, emb1, emb2, W1, W2):
    b, s = x.shape
    xp = x.T.reshape(-1)
    v = emb2.reshape(7500, 128, 8).transpose(0, 2, 1).reshape(60000, 128)
    e2lin = _sc_pack_e2(v)
    e2buf = _sc_gather_e2(xp, e2lin)
    t = _build_t(emb0, emb1, W1)
    Bmat = (
        jnp.eye(16, dtype=jnp.float32)[:, None, :, None] * W2[None, :, None, :]
    ).reshape(128, 2048)
    out0 = (
        _mm_e2(e2buf.reshape(N_TOK // 16, 128), Bmat)
        .reshape(N_TOK // 128, 8, 16, 128)
        .transpose(0, 2, 1, 3)
        .reshape(N_TOK, D_MODEL)
    )
    out_ref = jax.new_ref(out0)
    _sc_scatter_t(xp, t, out_ref)
    return out_ref[...].reshape(s, b, D_MODEL).transpose(1, 0, 2)


# speedup vs baseline: 17.6407x; 1.2845x over previous
import functools

import jax
import jax.numpy as jnp
from jax import lax
from jax.experimental import pallas as pl
from jax.experimental.pallas import tpu as pltpu
from jax.experimental.pallas import tpu_sc as plsc

VOCAB = 1_000_000
D_MODEL = 128
C0 = 20_000
C1 = 40_000
N_TOK = 4096 * 50
NC, NS, L = 2, 16, 16
NW = NC * NS
TPW = N_TOK // NW
G = 128
NGRP = TPW // G
NCGRP = TPW // G + 1

_SC_MESH = plsc.VectorSubcoreMesh(
    core_axis_name="c", subcore_axis_name="s", num_cores=NC, num_subcores=NS
)


@functools.partial(
    pl.kernel,
    out_type=jax.ShapeDtypeStruct((N_TOK, 8), jnp.float32),
    mesh=_SC_MESH,
    scratch_types=[
        pltpu.VMEM((TPW,), jnp.int32),
        pltpu.VMEM((NGRP, G), jnp.int32),
        pltpu.VMEM((TPW, 8), jnp.float32),
        pltpu.SemaphoreType.DMA,
    ],
    compiler_params=pltpu.CompilerParams(use_tc_tiling_on_sc=False, needs_layout_passes=False),
)
def _sc_gather_e2(x_hbm, emb2_hbm, e2buf_hbm, xv, idxv, rows, sem):
    wid = lax.axis_index("s") * NC + lax.axis_index("c")
    base = wid * TPW
    pltpu.sync_copy(x_hbm.at[pl.ds(base, TPW)], xv)

    lanes = lax.iota(jnp.int32, L)
    qbase = 16 * (lanes % 8) + lanes // 8

    @pl.loop(0, TPW // L)
    def _(g):
        xs = xv[pl.ds(g * L, L)]
        grp = jnp.full((L,), g // (G // L), jnp.int32)
        q = qbase + 2 * (g % (G // L))
        plsc.store_scatter(idxv, [grp, q], jnp.clip(xs - C1, 0, VOCAB - C1 - 1))

    @pl.loop(0, NGRP)
    def _(j):
        pltpu.async_copy(emb2_hbm.at[idxv.at[j]], rows.at[pl.ds(j * G, G)], sem)

    @pl.loop(0, NGRP)
    def _(j):
        pltpu.make_async_copy(
            emb2_hbm.at[idxv.at[j]], rows.at[pl.ds(j * G, G)], sem
        ).wait()

    pltpu.sync_copy(rows, e2buf_hbm.at[pl.ds(base, TPW)])


_PT = 25
_NSLAB = 7500 // _PT


@functools.partial(
    pl.kernel,
    out_type=jax.ShapeDtypeStruct((VOCAB - C1, 8), jnp.float32),
    mesh=_SC_MESH,
    scratch_types=[
        pltpu.VMEM((2, 8 * _PT, 128), jnp.float32),
        pltpu.VMEM((2, 128 * _PT, 8), jnp.float32),
        pltpu.SemaphoreType.DMA,
        pltpu.SemaphoreType.DMA,
    ],
    compiler_params=pltpu.CompilerParams(use_tc_tiling_on_sc=False, needs_layout_passes=False),
)
def _sc_pack_e2(v_hbm, o_hbm, vin, vout, isem, osem):
    wid = lax.axis_index("s") * NC + lax.axis_index("c")
    nmy = (_NSLAB - 1 - wid) // NW + 1
    lanes = lax.iota(jnp.int32, L)

    def _src(i):
        return v_hbm.at[pl.ds((wid + i * NW) * 8 * _PT, 8 * _PT)]

    def _dst(i):
        return o_hbm.at[pl.ds((wid + i * NW) * 128 * _PT, 128 * _PT)]

    pltpu.async_copy(_src(0), vin.at[0], isem)

    def _shuffle(vin_b, vout_b):
        @pl.loop(0, _PT)
        def _(t):
            @pl.loop(0, 64)
            def _(u):
                k, m = u // 8, u % 8
                val = vin_b[t * 8 + k, pl.ds(m * 16, L)]
                plsc.store_scatter(
                    vout_b,
                    [t * 128 + m * 16 + lanes, jnp.full((L,), k, jnp.int32)],
                    val,
                )

    def _slab(i, carry):
        b = lax.rem(i, 2)
        pltpu.make_async_copy(_src(i), vin.at[b], isem).wait()

        @pl.when(i + 1 < nmy)
        def _():
            pltpu.async_copy(_src(i + 1), vin.at[1 - b], isem)

        @pl.when(i >= 2)
        def _():
            pltpu.make_async_copy(vout.at[b], _dst(i - 2), osem).wait()

        @pl.when(b == 0)
        def _():
            _shuffle(vin.at[0], vout.at[0])

        @pl.when(b == 1)
        def _():
            _shuffle(vin.at[1], vout.at[1])

        pltpu.async_copy(vout.at[b], _dst(i), osem)
        return carry

    lax.fori_loop(0, nmy, _slab, jnp.int32(0))
    pltpu.make_async_copy(vout.at[lax.rem(nmy, 2)], _dst(nmy - 2), osem).wait()
    pltpu.make_async_copy(vout.at[lax.rem(nmy - 1, 2)], _dst(nmy - 1), osem).wait()


_TROWS = 400


def _build_t_body(emb0_ref, emb1_ref, w1_ref, t_ref):
    g = pl.program_id(0)

    @pl.when(g < 50)
    def _():
        t_ref[...] = emb0_ref[...]

    @pl.when(g >= 50)
    def _():
        t_ref[...] = jnp.dot(
            emb1_ref[...],
            w1_ref[...],
            preferred_element_type=jnp.float32,
            precision=lax.Precision.HIGHEST,
        )


def _build_t(emb0, emb1, W1):
    return pl.pallas_call(
        _build_t_body,
        grid=(100,),
        in_specs=[
            pl.BlockSpec((_TROWS, 128), lambda g: (jnp.minimum(g, 49), 0)),
            pl.BlockSpec((_TROWS, 32), lambda g: (jnp.maximum(g - 50, 0), 0)),
            pl.BlockSpec((32, 128), lambda g: (0, 0)),
        ],
        out_specs=pl.BlockSpec((_TROWS, 128), lambda g: (g, 0)),
        out_shape=jax.ShapeDtypeStruct((C1, 128), jnp.float32),
    )(emb0, emb1, W1)


_MROWS = 512


def _mm_body(p_ref, b_ref, o_ref):
    o_ref[...] = jnp.dot(
        p_ref[...],
        b_ref[...],
        preferred_element_type=jnp.float32,
        precision=lax.Precision.HIGHEST,
    )


def _mm_e2(e2packed, Bmat):
    return pl.pallas_call(
        _mm_body,
        grid=(N_TOK // 16 // _MROWS,),
        in_specs=[
            pl.BlockSpec((_MROWS, 128), lambda g: (g, 0)),
            pl.BlockSpec((128, 2048), lambda g: (0, 0)),
        ],
        out_specs=pl.BlockSpec((_MROWS, 2048), lambda g: (g, 0)),
        out_shape=jax.ShapeDtypeStruct((N_TOK // 16, 2048), jnp.float32),
    )(e2packed, Bmat)


@functools.partial(
    pl.kernel,
    out_type=(),
    mesh=_SC_MESH,
    scratch_types=[
        pltpu.VMEM((TPW,), jnp.int32),
        pltpu.VMEM((NCGRP, G), jnp.int32),
        pltpu.VMEM((NCGRP, G), jnp.int32),
        pltpu.VMEM((G, 128), jnp.float32),
        pltpu.SemaphoreType.DMA,
        pltpu.SemaphoreType.DMA,
    ],
    compiler_params=pltpu.CompilerParams(use_tc_tiling_on_sc=False, needs_layout_passes=False),
)
def _sc_scatter_t(x_hbm, t_hbm, out_hbm, xv, srcc, dstc, rows, gsem, ssem):
    wid = lax.axis_index("s") * NC + lax.axis_index("c")
    base = wid * TPW
    pltpu.sync_copy(x_hbm.at[pl.ds(base, TPW)], xv)

    def _compact(g, cnt):
        xs = xv[pl.ds(g * L, L)]
        m = xs < C1
        mi = jnp.where(m, 1, 0).astype(jnp.int32)
        pos = cnt + plsc.cumsum(mi) - 1
        src = jnp.minimum(xs, C1 - 1)
        dst = base + g * L + lax.iota(jnp.int32, L)
        plsc.store_scatter(srcc, [pos // G, pos % G], src, mask=m)
        plsc.store_scatter(dstc, [pos // G, pos % G], dst, mask=m)
        return cnt + jnp.sum(mi)

    n = lax.fori_loop(0, TPW // L, _compact, jnp.int32(0))

    lastp = jnp.maximum(n - 1, 0)
    lrow = jnp.full((L,), lastp // G, jnp.int32)
    lcol = jnp.full((L,), lastp % G, jnp.int32)
    lastsrc = plsc.load_gather(srcc, [lrow, lcol])
    lastdst = plsc.load_gather(dstc, [lrow, lcol])

    @pl.loop(0, G // L)
    def _(k):
        p = n + k * L + lax.iota(jnp.int32, L)
        plsc.store_scatter(srcc, [p // G, p % G], lastsrc)
        plsc.store_scatter(dstc, [p // G, p % G], lastdst)

    nsg = (n + G - 1) // G

    def _dma(j, carry):
        pltpu.async_copy(t_hbm.at[srcc.at[j]], rows, gsem)
        pltpu.make_async_copy(t_hbm.at[srcc.at[j]], rows, gsem).wait()
        pltpu.async_copy(rows, out_hbm.at[dstc.at[j]], ssem)
        pltpu.make_async_copy(rows, out_hbm.at[dstc.at[j]], ssem).wait()
        return carry

    lax.fori_loop(0, nsg, _dma, jnp.int32(0))


def kernel(x, emb0, emb1, emb2, W1, W2):
    b, s = x.shape
    xp = x.T.reshape(-1)
    v = emb2.reshape(7500, 128, 8).transpose(0, 2, 1).reshape(60000, 128)
    e2lin = _sc_pack_e2(v)
    e2buf = _sc_gather_e2(xp, e2lin)
    t = _build_t(emb0, emb1, W1)
    Bmat = (
        jnp.eye(16, dtype=jnp.float32)[:, None, :, None] * W2[None, :, None, :]
    ).reshape(128, 2048)
    out0 = (
        _mm_e2(e2buf.reshape(N_TOK // 16, 128), Bmat)
        .reshape(N_TOK // 128, 8, 16, 128)
        .transpose(0, 2, 1, 3)
        .reshape(N_TOK, D_MODEL)
    )
    out_ref = jax.new_ref(out0)
    _sc_scatter_t(xp, t, out_ref)
    return out_ref[...].reshape(s, b, D_MODEL).transpose(1, 0, 2)
